# baseline (speedup 1.0000x reference)
import functools

import jax
import jax.numpy as jnp
import numpy as np
from jax import lax
from jax.experimental import pallas as pl
from jax.experimental.pallas import tpu as pltpu
from jax.experimental.pallas import tpu_sc as plsc

D = 8
DP = 16
DD = D * D
NC = 2
NS = 16
NW = NC * NS
CHUNK = 400
STAGE = 1250

_GDN = lax.GatherDimensionNumbers(
    offset_dims=(), collapsed_slice_dims=(0,), start_index_map=(0,)
)


def _lane_shuffle(v, idx):
    return lax.gather(
        v, idx.reshape(16, 1), _GDN, slice_sizes=(1,),
        mode=lax.GatherScatterMode.PROMISE_IN_BOUNDS,
    )


def _make(n_edges, n_nodes):
    e_per_w = n_edges // NW
    n_chunks = e_per_w // CHUNK
    rows_per_tile = n_nodes // NS
    n_stage = rows_per_tile // STAGE
    mesh = plsc.VectorSubcoreMesh(core_axis_name="c", subcore_axis_name="s")

    def body(si_hbm, ri_hbm, s_flat, r_flat,
             out_hbm, s_tab, r_tab,
             stage_in, stage_out, sidx_v, ridx_v, srow_v, rrow_v, out_v,
             sem_st, sem_si, sem_ri, sem_sg, sem_rg, sem_out):
        sid = lax.axis_index("s")
        wid = sid * NC + lax.axis_index("c")
        base_w = wid * e_per_w
        lane = lax.iota(jnp.int32, 16)
        s_pat_idx = [lane // D + 2 * k for k in range(4)]
        dup_a = lane % D
        dup_b = D + lane % D

        def build(tab_flat, tab_out):
            def step(j, carry):
                row0 = sid * rows_per_tile + j * STAGE
                pltpu.make_async_copy(
                    tab_flat.at[pl.ds(row0 * D, STAGE * D)], stage_in, sem_st
                ).start()
                pltpu.make_async_copy(
                    tab_flat.at[pl.ds(0, STAGE * D)], stage_in, sem_st
                ).wait()

                def pair(p, c2):
                    v = stage_in[pl.ds(p * 16, 16)]
                    stage_out[2 * p] = _lane_shuffle(v, dup_a)
                    stage_out[2 * p + 1] = _lane_shuffle(v, dup_b)
                    return c2

                lax.fori_loop(0, STAGE // 2, pair, 0, unroll=4)
                pltpu.sync_copy(stage_out, tab_out.at[pl.ds(row0, STAGE)])
                return carry

            lax.fori_loop(0, n_stage, step, 0, unroll=False)

        build(s_flat, s_tab)
        build(r_flat, r_tab)
        plsc.subcore_barrier()

        def idx_start(k, b):
            base = base_w + k * CHUNK
            pltpu.make_async_copy(
                si_hbm.at[pl.ds(base, CHUNK)], sidx_v.at[b], sem_si.at[b]
            ).start()
            pltpu.make_async_copy(
                ri_hbm.at[pl.ds(base, CHUNK)], ridx_v.at[b], sem_ri.at[b]
            ).start()

        def idx_wait(b):
            pltpu.make_async_copy(
                si_hbm.at[pl.ds(0, CHUNK)], sidx_v.at[b], sem_si.at[b]
            ).wait()
            pltpu.make_async_copy(
                ri_hbm.at[pl.ds(0, CHUNK)], ridx_v.at[b], sem_ri.at[b]
            ).wait()

        def gather_start(b):
            pltpu.make_async_copy(
                s_tab.at[sidx_v.at[b]], srow_v.at[b], sem_sg.at[b]
            ).start()
            pltpu.make_async_copy(
                r_tab.at[ridx_v.at[b]], rrow_v.at[b], sem_rg.at[b]
            ).start()

        def gather_wait(b):
            pltpu.make_async_copy(
                s_tab.at[sidx_v.at[b]], srow_v.at[b], sem_sg.at[b]
            ).wait()
            pltpu.make_async_copy(
                r_tab.at[ridx_v.at[b]], rrow_v.at[b], sem_rg.at[b]
            ).wait()

        def out_start(k, b):
            base = base_w + k * CHUNK
            pltpu.make_async_copy(
                out_v.at[b], out_hbm.at[pl.ds(base * DD, CHUNK * DD)], sem_out.at[b]
            ).start()

        def out_wait(b):
            pltpu.make_async_copy(
                out_v.at[b], out_hbm.at[pl.ds(0, CHUNK * DD)], sem_out.at[b]
            ).wait()

        def compute(b):
            def emit(c, s_vec, r_rep):
                c64 = c * DD
                for k in range(4):
                    s_pat = _lane_shuffle(s_vec, s_pat_idx[k])
                    out_v[b, pl.ds(c64 + k * 16, 16)] = s_pat * r_rep

            def edge_body(c, carry):
                s_vec, r_rep = carry
                nxt = (srow_v[b, c + 1], rrow_v[b, c + 1])
                emit(c, s_vec, r_rep)
                return nxt

            first = (srow_v[b, 0], rrow_v[b, 0])
            last = lax.fori_loop(0, CHUNK - 1, edge_body, first, unroll=8)
            emit(CHUNK - 1, *last)

        idx_start(0, 0)
        idx_start(1, 1)
        idx_wait(0)
        gather_start(0)

        def pair_body(k2, carry):
            for b in (0, 1):
                k = 2 * k2 + b

                @pl.when(k < n_chunks)
                def _():
                    @pl.when(k + 1 < n_chunks)
                    def _():
                        idx_wait(1 - b)
                        gather_start(1 - b)

                    gather_wait(b)

                    @pl.when(k + 2 < n_chunks)
                    def _():
                        idx_start(k + 2, b)

                    @pl.when(k >= 2)
                    def _():
                        out_wait(b)

                    compute(b)
                    out_start(k, b)

            return carry

        lax.fori_loop(0, (n_chunks + 1) // 2, pair_body, 0, unroll=False)
        out_wait(0)
        out_wait(1)

    return pl.kernel(
        body,
        out_type=(
            jax.ShapeDtypeStruct((n_edges * DD,), jnp.float32),
            jax.ShapeDtypeStruct((n_nodes, DP), jnp.float32),
            jax.ShapeDtypeStruct((n_nodes, DP), jnp.float32),
        ),
        mesh=mesh,
        compiler_params=pltpu.CompilerParams(
            needs_layout_passes=False, use_tc_tiling_on_sc=False
        ),
        scratch_types=[
            pltpu.VMEM((STAGE * D,), jnp.float32),
            pltpu.VMEM((STAGE, DP), jnp.float32),
            pltpu.VMEM((2, CHUNK), jnp.int32),
            pltpu.VMEM((2, CHUNK), jnp.int32),
            pltpu.VMEM((2, CHUNK, DP), jnp.float32),
            pltpu.VMEM((2, CHUNK, DP), jnp.float32),
            pltpu.VMEM((2, CHUNK * DD), jnp.float32),
            pltpu.SemaphoreType.DMA,
            pltpu.SemaphoreType.DMA((2,)),
            pltpu.SemaphoreType.DMA((2,)),
            pltpu.SemaphoreType.DMA((2,)),
            pltpu.SemaphoreType.DMA((2,)),
            pltpu.SemaphoreType.DMA((2,)),
        ],
    )


def kernel(edge_index, node_type_s, node_type_r=None):
    if node_type_r is None:
        node_type_r = node_type_s
    n_edges = edge_index.shape[1]
    n_nodes = node_type_s.shape[0]
    f = _make(n_edges, n_nodes)
    out_flat, _, _ = f(
        edge_index[0], edge_index[1],
        node_type_s.reshape(-1), node_type_r.reshape(-1),
    )
    return out_flat.reshape(n_edges, DD)
